# baseline (speedup 1.0000x reference)
import functools

import jax
import jax.numpy as jnp
from jax import lax
from jax.experimental import pallas as pl
from jax.experimental.pallas import tpu as pltpu
from jax.experimental.pallas import tpu_sc as plsc

N_NODES = 25000
D = 64
E = 800000
CHUNK = 96
CPT = 522
E_PAD = 16 * CPT * CHUNK
ROWS2 = E_PAD // CHUNK
BANK = 9
NBANKS = CPT // BANK
NBUF = 4
FBUF = 2
G = 2
ZROWS = 96
NZFULL = N_NODES // ZROWS
ZREM = N_NODES - NZFULL * ZROWS
ZITERS = (NZFULL + 15) // 16

_ILV = plsc.PackFormat.INTERLEAVED


def _edge_loop(sid, dst_hbm, src_hbm, vals_hbm, gtable_hbm, acc,
               didx, sidx, vbank, gring, fring, gsem, ssem, isem):
    base_row = sid * CPT

    pltpu.sync_copy(dst_hbm.at[pl.ds(base_row, BANK)], didx.at[0])
    pltpu.sync_copy(src_hbm.at[pl.ds(base_row, BANK)], sidx.at[0])
    pltpu.sync_copy(vals_hbm.at[pl.ds(base_row, BANK)], vbank.at[0])
    for pj in range(G):
        pltpu.async_copy(gtable_hbm.at[sidx.at[0, pj]],
                         gring.at[pj], gsem.at[pj])

    def edge_chunk(j, carry):
        jb, b, rg, bg = carry
        p = b & 1
        pg = bg & 1
        slot = j & (NBUF - 1)
        fslot = j & (FBUF - 1)

        @pl.when(jnp.logical_and(jb == BANK - G, b < NBANKS - 1))
        def _():
            for _k in range(3):
                pltpu.make_async_copy(
                    dst_hbm.at[pl.ds(base_row, BANK)],
                    didx.at[1 - p], isem).wait()

        pltpu.make_async_copy(gtable_hbm.at[sidx.at[p, jb]],
                              gring.at[slot], gsem.at[slot]).wait()

        @pl.when(j >= FBUF)
        def _():
            pltpu.make_async_copy(fring.at[fslot],
                                  acc.at[didx.at[0, 0]],
                                  ssem.at[fslot]).wait()

        for g2 in range(CHUNK // 16):
            vv = vbank[p, jb, pl.ds(g2 * 16, 16)]
            for l in range(16):
                k = g2 * 16 + l
                v = vv[l]
                for h in range(2):
                    ab = gring[slot, k, pl.ds(h * 32, 32)]
                    a, b2 = plsc.unpack(ab, format=_ILV)
                    fring[fslot, k, pl.ds(h * 32, 16)] = a * v
                    fring[fslot, k, pl.ds(h * 32 + 16, 16)] = b2 * v

        pltpu.async_copy(fring.at[fslot], acc.at[didx.at[p, jb]],
                         ssem.at[fslot], add=True)

        @pl.when(j + G < CPT)
        def _():
            pltpu.async_copy(gtable_hbm.at[sidx.at[pg, rg]],
                             gring.at[(j + G) & (NBUF - 1)],
                             gsem.at[(j + G) & (NBUF - 1)])

        @pl.when(jnp.logical_and(jb == 1, b < NBANKS - 1))
        def _():
            off = base_row + (b + 1) * BANK
            pltpu.async_copy(dst_hbm.at[pl.ds(off, BANK)],
                             didx.at[1 - p], isem)
            pltpu.async_copy(src_hbm.at[pl.ds(off, BANK)],
                             sidx.at[1 - p], isem)
            pltpu.async_copy(vals_hbm.at[pl.ds(off, BANK)],
                             vbank.at[1 - p], isem)

        jb = jb + 1
        wrap = jb == BANK
        b = jnp.where(wrap, b + 1, b)
        jb = jnp.where(wrap, 0, jb)
        rg = rg + 1
        wrapg = rg == BANK
        bg = jnp.where(wrapg, bg + 1, bg)
        rg = jnp.where(wrapg, 0, rg)
        return (jb, b, rg, bg)

    lax.fori_loop(0, CPT, edge_chunk,
                  (jnp.int32(0), jnp.int32(0),
                   jnp.int32(G), jnp.int32(0)))

    for _k in range(FBUF):
        s = (CPT - FBUF + _k) & (FBUF - 1)
        pltpu.make_async_copy(fring.at[s], acc.at[didx.at[0, 0]],
                              ssem.at[s]).wait()


def _zero_acc(sid, acc, fring):
    def zero_buf(r, c):
        for j in range(4):
            fring[0, r, pl.ds(j * 16, 16)] = jnp.zeros((16,), jnp.float32)
        return c
    lax.fori_loop(0, ZROWS, zero_buf, 0)

    def zero_chunk(it, c):
        cid = it * 16 + sid

        @pl.when(cid < NZFULL)
        def _():
            pltpu.sync_copy(fring.at[0], acc.at[pl.ds(cid * ZROWS, ZROWS)])
        return c
    lax.fori_loop(0, ZITERS, zero_chunk, 0)

    @pl.when(sid == 0)
    def _():
        pltpu.sync_copy(fring.at[0].at[pl.ds(0, ZREM)],
                        acc.at[pl.ds(NZFULL * ZROWS, ZREM)])


def _mid_body(rows_hbm, cols_hbm, vals_hbm, ub_hbm, ib_hbm,
              new_ub, new_ib, acc, didx, sidx, vbank, gring, fring,
              gsem, ssem, isem):
    sid = lax.axis_index("s")
    core = lax.axis_index("c")

    def run_direction(dst_hbm, src_hbm, gtable_hbm, out_hbm):
        _zero_acc(sid, acc, fring)
        plsc.subcore_barrier()
        _edge_loop(sid, dst_hbm, src_hbm, vals_hbm, gtable_hbm, acc,
                   didx, sidx, vbank, gring, fring, gsem, ssem, isem)
        plsc.subcore_barrier()

        def pack_rows(nrows):
            def row(r, c2):
                for h in range(2):
                    a = fring[0, r, pl.ds(h * 32, 16)]
                    b2 = fring[0, r, pl.ds(h * 32 + 16, 16)]
                    gring[0, r, pl.ds(h * 32, 32)] = plsc.pack(
                        a, b2, format=_ILV)
                return c2
            lax.fori_loop(0, nrows, row, 0)

        def export(it, c):
            cid = it * 16 + sid

            @pl.when(cid < NZFULL)
            def _():
                sl = pl.ds(cid * ZROWS, ZROWS)
                pltpu.sync_copy(acc.at[sl], fring.at[0])
                pack_rows(ZROWS)
                pltpu.sync_copy(gring.at[0], out_hbm.at[sl])
            return c
        lax.fori_loop(0, ZITERS, export, 0)

        @pl.when(sid == 0)
        def _():
            sl = pl.ds(NZFULL * ZROWS, ZREM)
            pltpu.sync_copy(acc.at[sl], fring.at[0].at[pl.ds(0, ZREM)])
            pack_rows(ZREM)
            pltpu.sync_copy(gring.at[0].at[pl.ds(0, ZREM)], out_hbm.at[sl])

    @pl.when(core == 0)
    def _():
        run_direction(rows_hbm, cols_hbm, ib_hbm, new_ub)

    @pl.when(core == 1)
    def _():
        run_direction(cols_hbm, rows_hbm, ub_hbm, new_ib)


def _last_body(rows_hbm, cols_hbm, vals_hbm, ub_hbm, ib_hbm,
               u0_hbm, i0_hbm, u1b_hbm, i1b_hbm,
               pref_u, pref_i, acc, didx, sidx, vbank, gring, fring,
               gsem, ssem, isem):
    sid = lax.axis_index("s")
    core = lax.axis_index("c")

    def run_direction(dst_hbm, src_hbm, gtable_hbm, out_hbm,
                      t0_hbm, t1b_hbm, t2b_hbm):
        _zero_acc(sid, acc, fring)
        plsc.subcore_barrier()
        _edge_loop(sid, dst_hbm, src_hbm, vals_hbm, gtable_hbm, acc,
                   didx, sidx, vbank, gring, fring, gsem, ssem, isem)
        plsc.subcore_barrier()

        def sum_rows(nrows):
            def row(r, c2):
                for h in range(2):
                    a1, b1 = plsc.unpack(gring[0, r, pl.ds(h * 32, 32)],
                                         format=_ILV)
                    a2, b2 = plsc.unpack(gring[1, r, pl.ds(h * 32, 32)],
                                         format=_ILV)
                    slo = pl.ds(h * 32, 16)
                    shi = pl.ds(h * 32 + 16, 16)
                    lo = (fring[0, r, slo] + fring[1, r, slo]
                          + a1 + a2) * 0.25
                    hi = (fring[0, r, shi] + fring[1, r, shi]
                          + b1 + b2) * 0.25
                    fring[0, r, slo] = lo
                    fring[0, r, shi] = hi
                return c2
            lax.fori_loop(0, nrows, row, 0)

        def export(it, c):
            cid = it * 16 + sid

            @pl.when(cid < NZFULL)
            def _():
                sl = pl.ds(cid * ZROWS, ZROWS)
                pltpu.sync_copy(acc.at[sl], fring.at[0])
                pltpu.sync_copy(t0_hbm.at[sl], fring.at[1])
                pltpu.sync_copy(t1b_hbm.at[sl], gring.at[0])
                pltpu.sync_copy(t2b_hbm.at[sl], gring.at[1])
                sum_rows(ZROWS)
                pltpu.sync_copy(fring.at[0], out_hbm.at[sl])
            return c
        lax.fori_loop(0, ZITERS, export, 0)

        @pl.when(sid == 0)
        def _():
            sl = pl.ds(NZFULL * ZROWS, ZREM)
            zr = pl.ds(0, ZREM)
            pltpu.sync_copy(acc.at[sl], fring.at[0].at[zr])
            pltpu.sync_copy(t0_hbm.at[sl], fring.at[1].at[zr])
            pltpu.sync_copy(t1b_hbm.at[sl], gring.at[0].at[zr])
            pltpu.sync_copy(t2b_hbm.at[sl], gring.at[1].at[zr])
            sum_rows(ZREM)
            pltpu.sync_copy(fring.at[0].at[zr], out_hbm.at[sl])

    @pl.when(core == 0)
    def _():
        run_direction(rows_hbm, cols_hbm, ib_hbm, pref_u,
                      u0_hbm, u1b_hbm, ub_hbm)

    @pl.when(core == 1)
    def _():
        run_direction(cols_hbm, rows_hbm, ub_hbm, pref_i,
                      i0_hbm, i1b_hbm, ib_hbm)


_SC_SCRATCH = [
    pltpu.VMEM_SHARED((N_NODES, D), jnp.float32),
    pltpu.VMEM((2, BANK, CHUNK), jnp.int32),
    pltpu.VMEM((2, BANK, CHUNK), jnp.int32),
    pltpu.VMEM((2, BANK, CHUNK), jnp.float32),
    pltpu.VMEM((NBUF, CHUNK, D), jnp.bfloat16),
    pltpu.VMEM((FBUF, CHUNK, D), jnp.float32),
    pltpu.SemaphoreType.DMA((NBUF,)),
    pltpu.SemaphoreType.DMA((FBUF,)),
    pltpu.SemaphoreType.DMA,
]

_MESH = plsc.VectorSubcoreMesh(core_axis_name="c", subcore_axis_name="s")

_propagate_mid = functools.partial(
    pl.kernel,
    out_type=(jax.ShapeDtypeStruct((N_NODES, D), jnp.bfloat16),
              jax.ShapeDtypeStruct((N_NODES, D), jnp.bfloat16)),
    mesh=_MESH,
    scratch_types=_SC_SCRATCH,
    compiler_params=pltpu.CompilerParams(use_tc_tiling_on_sc=False,
                                         needs_layout_passes=False),
)(_mid_body)

_propagate_last = functools.partial(
    pl.kernel,
    out_type=(jax.ShapeDtypeStruct((N_NODES, D), jnp.float32),
              jax.ShapeDtypeStruct((N_NODES, D), jnp.float32)),
    mesh=_MESH,
    scratch_types=_SC_SCRATCH,
    compiler_params=pltpu.CompilerParams(use_tc_tiling_on_sc=False,
                                         needs_layout_passes=False),
)(_last_body)


ROWS_BLK = 1000


def _matmul_body(us, vs, umap, imap, su, si):
    su[...] = jnp.dot(us[...], umap[...], preferred_element_type=jnp.float32)
    si[...] = jnp.dot(vs[...], imap[...], preferred_element_type=jnp.float32)


def _structure(us, vs, umap, imap):
    row_spec = pl.BlockSpec((ROWS_BLK, D), lambda i: (i, 0))
    map_spec = pl.BlockSpec((D, D), lambda i: (0, 0))
    return pl.pallas_call(
        _matmul_body,
        grid=(N_NODES // ROWS_BLK,),
        in_specs=[row_spec, row_spec, map_spec, map_spec],
        out_specs=(row_spec, row_spec),
        out_shape=(jax.ShapeDtypeStruct((N_NODES, D), jnp.float32),
                   jax.ShapeDtypeStruct((N_NODES, D), jnp.float32)),
    )(us, vs, umap, imap)


def _interleave_bf16(x):
    n = x.shape[0]
    return (x.reshape(n, 2, 2, 16).transpose(0, 1, 3, 2)
            .reshape(n, D).astype(jnp.bfloat16))


def kernel(edge_index, edge_vals, user_preference, item_preference,
           user_map, item_map, U_mul_S, V_mul_S):
    rows = edge_index[0].astype(jnp.int32)
    cols = edge_index[1].astype(jnp.int32)
    vals = edge_vals.astype(jnp.float32)

    pad = E_PAD - E
    rows2 = jnp.concatenate(
        [rows, jnp.zeros((pad,), jnp.int32)]).reshape(ROWS2, CHUNK)
    cols2 = jnp.concatenate(
        [cols, jnp.zeros((pad,), jnp.int32)]).reshape(ROWS2, CHUNK)
    vals2 = jnp.concatenate(
        [vals, jnp.zeros((pad,), jnp.float32)]).reshape(ROWS2, CHUNK)

    u0b = _interleave_bf16(user_preference)
    i0b = _interleave_bf16(item_preference)

    u1b, i1b = _propagate_mid(rows2, cols2, vals2, u0b, i0b)
    u2b, i2b = _propagate_mid(rows2, cols2, vals2, u1b, i1b)
    pref_u, pref_i = _propagate_last(rows2, cols2, vals2, u2b, i2b,
                                     user_preference, item_preference,
                                     u1b, i1b)

    su, si = _structure(U_mul_S, V_mul_S, user_map, item_map)
    return jnp.stack([pref_u, pref_i, su, si], axis=0)
